# baseline (speedup 1.0000x reference)
import functools

import jax
import jax.numpy as jnp
from jax import lax
from jax.experimental import pallas as pl
from jax.experimental.pallas import tpu as pltpu
from jax.experimental.pallas import tpu_sc as plsc

N_NODES = 10000
N_EDGES = 320000
D_IN = 128
D_EXT = 144

NC = 2
NS = 16
NW = NC * NS

CHUNK = 64
N_CHUNKS = 158
EDGES_PER_TILE = CHUNK * N_CHUNKS
PAD_PER_TILE = EDGES_PER_TILE - N_EDGES // NW

ACC_ROWS = 10112
ZERO_ROWS = ACC_ROWS // NS


def _sc_segment_sum(x_ext, eidx, zrows):
  mesh = plsc.VectorSubcoreMesh(core_axis_name="c", subcore_axis_name="s")

  @functools.partial(
      pl.kernel,
      out_type=jax.ShapeDtypeStruct((NC, ACC_ROWS, D_EXT), jnp.float32),
      mesh=mesh,
      scratch_types=[
          pltpu.VMEM((N_CHUNKS, CHUNK), jnp.int32),
          pltpu.VMEM((N_CHUNKS, CHUNK), jnp.int32),
          pltpu.VMEM((CHUNK, D_EXT), jnp.float32),
          pltpu.VMEM((CHUNK, D_EXT), jnp.float32),
          pltpu.VMEM_SHARED((ACC_ROWS, D_EXT), jnp.float32),
          pltpu.SemaphoreType.DMA,
          pltpu.SemaphoreType.DMA,
      ],
      compiler_params=pltpu.CompilerParams(use_tc_tiling_on_sc=False),
  )
  def k(xext_hbm, eidx_hbm, zrows_hbm, out_hbm, src_v, dst_v, rows0, rows1,
        acc, gsem, ssem):
    c = lax.axis_index("c")
    s = lax.axis_index("s")
    w = c * NS + s

    pltpu.sync_copy(eidx_hbm.at[0, w], src_v)
    pltpu.sync_copy(eidx_hbm.at[1, w], dst_v)

    pltpu.sync_copy(zrows_hbm, acc.at[pl.ds(s * ZERO_ROWS, ZERO_ROWS)])
    plsc.subcore_barrier()

    def body(g, carry):
      a = 2 * g
      b = a + 1
      d0 = pltpu.async_copy(xext_hbm.at[src_v.at[a]], rows0, gsem)
      d1 = pltpu.async_copy(xext_hbm.at[src_v.at[b]], rows1, gsem)
      d0.wait()
      d1.wait()
      e0 = pltpu.async_copy(rows0, acc.at[dst_v.at[a]], ssem, add=True)
      e1 = pltpu.async_copy(rows1, acc.at[dst_v.at[b]], ssem, add=True)
      e0.wait()
      e1.wait()
      return carry

    lax.fori_loop(0, N_CHUNKS // 2, body, 0)
    plsc.subcore_barrier()

    base = s * ZERO_ROWS
    pltpu.sync_copy(acc.at[pl.ds(base, ZERO_ROWS)],
                    out_hbm.at[c, pl.ds(base, ZERO_ROWS)])

  return k(x_ext, eidx, zrows)


ROWS_BLK = 1000


def _tc_body(p_ref, x_ref, wlt_ref, bl_ref, wrt_ref, wfct_ref, bfc_ref,
             h_ref, probs_ref):
  p = p_ref[0] + p_ref[1]
  summed = p[:, :D_IN]
  cnt = p[:, D_IN:D_IN + 1]
  mean = summed / jnp.maximum(cnt, 1.0)
  h = (jnp.dot(mean, wlt_ref[...], preferred_element_type=jnp.float32)
       + bl_ref[...]
       + jnp.dot(x_ref[...], wrt_ref[...], preferred_element_type=jnp.float32))
  h_ref[...] = h
  logits = jnp.dot(h, wfct_ref[...], preferred_element_type=jnp.float32)
  probs_ref[...] = jax.nn.sigmoid(logits + bfc_ref[...])


def _tc_head(partials, x, W_lT, b_l, W_rT, W_fcT, b_fc2):
  n_blocks = N_NODES // ROWS_BLK
  return pl.pallas_call(
      _tc_body,
      grid=(n_blocks,),
      in_specs=[
          pl.BlockSpec((NC, ROWS_BLK, D_EXT), lambda i: (0, i, 0)),
          pl.BlockSpec((ROWS_BLK, D_IN), lambda i: (i, 0)),
          pl.BlockSpec((D_IN, D_IN), lambda i: (0, 0)),
          pl.BlockSpec((1, D_IN), lambda i: (0, 0)),
          pl.BlockSpec((D_IN, D_IN), lambda i: (0, 0)),
          pl.BlockSpec((D_IN, 1), lambda i: (0, 0)),
          pl.BlockSpec((1, 1), lambda i: (0, 0)),
      ],
      out_specs=[
          pl.BlockSpec((ROWS_BLK, D_IN), lambda i: (i, 0)),
          pl.BlockSpec((ROWS_BLK, 1), lambda i: (i, 0)),
      ],
      out_shape=[
          jax.ShapeDtypeStruct((N_NODES, D_IN), jnp.float32),
          jax.ShapeDtypeStruct((N_NODES, 1), jnp.float32),
      ],
  )(partials, x, W_lT, b_l, W_rT, W_fcT, b_fc2)


def kernel(x, edge_index, W_l, b_l, W_r, W_fc, b_fc):
  x_ext = jnp.concatenate(
      [x, jnp.ones((N_NODES, D_EXT - D_IN), dtype=jnp.float32)], axis=1)
  ei = edge_index.reshape(2, NW, N_EDGES // NW)
  pad_src = jnp.zeros((1, NW, PAD_PER_TILE), dtype=jnp.int32)
  pad_dst = jnp.broadcast_to(
      N_NODES + jnp.arange(PAD_PER_TILE, dtype=jnp.int32) % (ACC_ROWS - N_NODES),
      (1, NW, PAD_PER_TILE))
  eidx = jnp.concatenate([ei, jnp.concatenate([pad_src, pad_dst], 0)], axis=2)
  eidx = eidx.reshape(2, NW, N_CHUNKS, CHUNK)
  zrows = jnp.zeros((ZERO_ROWS, D_EXT), dtype=jnp.float32)

  partials = _sc_segment_sum(x_ext, eidx, zrows)

  h, probs = _tc_head(partials, x, W_l.T, b_l.reshape(1, D_IN),
                      W_r.T, W_fc.T, b_fc.reshape(1, 1))
  return (h, probs.reshape(N_NODES))


# speedup vs baseline: 1.0502x; 1.0502x over previous
import functools

import jax
import jax.numpy as jnp
from jax import lax
from jax.experimental import pallas as pl
from jax.experimental.pallas import tpu as pltpu
from jax.experimental.pallas import tpu_sc as plsc

N_NODES = 10000
N_EDGES = 320000
D_IN = 128
D_EXT = 144

NC = 2
NS = 16
NW = NC * NS

CHUNK = 40
N_CHUNKS = 250
EDGES_PER_TILE = CHUNK * N_CHUNKS
PAD_PER_TILE = EDGES_PER_TILE - N_EDGES // NW

ACC_ROWS = 10112
ZERO_ROWS = ACC_ROWS // NS


def _sc_segment_sum(x_ext, eidx, zrows):
  mesh = plsc.VectorSubcoreMesh(core_axis_name="c", subcore_axis_name="s")

  @functools.partial(
      pl.kernel,
      out_type=jax.ShapeDtypeStruct((NC, ACC_ROWS, D_EXT), jnp.float32),
      mesh=mesh,
      scratch_types=[
          pltpu.VMEM((N_CHUNKS, CHUNK), jnp.int32),
          pltpu.VMEM((N_CHUNKS, CHUNK), jnp.int32),
          pltpu.VMEM((CHUNK, D_EXT), jnp.float32),
          pltpu.VMEM_SHARED((ACC_ROWS, D_EXT), jnp.float32),
          pltpu.SemaphoreType.DMA,
      ],
      compiler_params=pltpu.CompilerParams(use_tc_tiling_on_sc=False),
  )
  def k(xext_hbm, eidx_hbm, zrows_hbm, out_hbm, src_v, dst_v, rows0,
        acc, gsem):
    c = lax.axis_index("c")
    s = lax.axis_index("s")
    w = c * NS + s

    pltpu.sync_copy(eidx_hbm.at[0, w], src_v)
    pltpu.sync_copy(eidx_hbm.at[1, w], dst_v)

    pltpu.sync_copy(zrows_hbm, acc.at[pl.ds(s * ZERO_ROWS, ZERO_ROWS)])
    plsc.subcore_barrier()

    def body(j, carry):
      pltpu.async_copy(xext_hbm.at[src_v.at[j]], rows0, gsem).wait()
      pltpu.sync_copy(rows0, acc.at[dst_v.at[j]], add=True)
      return carry

    lax.fori_loop(0, N_CHUNKS, body, 0)
    plsc.subcore_barrier()

    base = s * ZERO_ROWS
    pltpu.sync_copy(acc.at[pl.ds(base, ZERO_ROWS)],
                    out_hbm.at[c, pl.ds(base, ZERO_ROWS)])

  return k(x_ext, eidx, zrows)


ROWS_BLK = 1000


def _tc_body(p_ref, x_ref, wlt_ref, bl_ref, wrt_ref, wfct_ref, bfc_ref,
             h_ref, probs_ref):
  p = p_ref[0] + p_ref[1]
  summed = p[:, :D_IN]
  cnt = p[:, D_IN:D_IN + 1]
  mean = summed / jnp.maximum(cnt, 1.0)
  h = (jnp.dot(mean, wlt_ref[...], preferred_element_type=jnp.float32)
       + bl_ref[...]
       + jnp.dot(x_ref[...], wrt_ref[...], preferred_element_type=jnp.float32))
  h_ref[...] = h
  logits = jnp.dot(h, wfct_ref[...], preferred_element_type=jnp.float32)
  probs_ref[...] = jax.nn.sigmoid(logits + bfc_ref[...])


def _tc_head(partials, x, W_lT, b_l, W_rT, W_fcT, b_fc2):
  n_blocks = N_NODES // ROWS_BLK
  return pl.pallas_call(
      _tc_body,
      grid=(n_blocks,),
      in_specs=[
          pl.BlockSpec((NC, ROWS_BLK, D_EXT), lambda i: (0, i, 0)),
          pl.BlockSpec((ROWS_BLK, D_IN), lambda i: (i, 0)),
          pl.BlockSpec((D_IN, D_IN), lambda i: (0, 0)),
          pl.BlockSpec((1, D_IN), lambda i: (0, 0)),
          pl.BlockSpec((D_IN, D_IN), lambda i: (0, 0)),
          pl.BlockSpec((D_IN, 1), lambda i: (0, 0)),
          pl.BlockSpec((1, 1), lambda i: (0, 0)),
      ],
      out_specs=[
          pl.BlockSpec((ROWS_BLK, D_IN), lambda i: (i, 0)),
          pl.BlockSpec((ROWS_BLK, 1), lambda i: (i, 0)),
      ],
      out_shape=[
          jax.ShapeDtypeStruct((N_NODES, D_IN), jnp.float32),
          jax.ShapeDtypeStruct((N_NODES, 1), jnp.float32),
      ],
  )(partials, x, W_lT, b_l, W_rT, W_fcT, b_fc2)


def kernel(x, edge_index, W_l, b_l, W_r, W_fc, b_fc):
  x_ext = jnp.concatenate(
      [x, jnp.ones((N_NODES, D_EXT - D_IN), dtype=jnp.float32)], axis=1)
  ei = edge_index.reshape(2, NW, N_EDGES // NW)
  pad_src = jnp.zeros((1, NW, PAD_PER_TILE), dtype=jnp.int32)
  pad_dst = jnp.broadcast_to(
      N_NODES + jnp.arange(PAD_PER_TILE, dtype=jnp.int32) % (ACC_ROWS - N_NODES),
      (1, NW, PAD_PER_TILE))
  eidx = jnp.concatenate([ei, jnp.concatenate([pad_src, pad_dst], 0)], axis=2)
  eidx = eidx.reshape(2, NW, N_CHUNKS, CHUNK)
  zrows = jnp.zeros((ZERO_ROWS, D_EXT), dtype=jnp.float32)

  partials = _sc_segment_sum(x_ext, eidx, zrows)

  h, probs = _tc_head(partials, x, W_l.T, b_l.reshape(1, D_IN),
                      W_r.T, W_fc.T, b_fc.reshape(1, 1))
  return (h, probs.reshape(N_NODES))


# speedup vs baseline: 1.3396x; 1.2756x over previous
import functools

import jax
import jax.numpy as jnp
from jax import lax
from jax.experimental import pallas as pl
from jax.experimental.pallas import tpu as pltpu
from jax.experimental.pallas import tpu_sc as plsc

N_NODES = 10000
N_EDGES = 320000
D_IN = 128
D_EXT = 144

NC = 2
NS = 16
NW = NC * NS

CHUNK = 80
N_CHUNKS = 125
EDGES_PER_TILE = CHUNK * N_CHUNKS
PAD_PER_TILE = EDGES_PER_TILE - N_EDGES // NW

ACC_ROWS = 10112
ZERO_ROWS = ACC_ROWS // NS


def _sc_segment_sum(x_ext, eidx, zrows):
  mesh = plsc.VectorSubcoreMesh(core_axis_name="c", subcore_axis_name="s")

  @functools.partial(
      pl.kernel,
      out_type=jax.ShapeDtypeStruct((NC, ACC_ROWS, D_EXT), jnp.float32),
      mesh=mesh,
      scratch_types=[
          pltpu.VMEM((N_CHUNKS, CHUNK), jnp.int32),
          pltpu.VMEM((N_CHUNKS, CHUNK), jnp.int32),
          pltpu.VMEM((CHUNK, D_EXT), jnp.float32),
          pltpu.VMEM_SHARED((ACC_ROWS, D_EXT), jnp.float32),
          pltpu.SemaphoreType.DMA,
      ],
      compiler_params=pltpu.CompilerParams(use_tc_tiling_on_sc=False),
  )
  def k(xext_hbm, eidx_hbm, zrows_hbm, out_hbm, src_v, dst_v, rows0,
        acc, gsem):
    c = lax.axis_index("c")
    s = lax.axis_index("s")
    w = c * NS + s

    pltpu.sync_copy(eidx_hbm.at[0, w], src_v)
    pltpu.sync_copy(eidx_hbm.at[1, w], dst_v)

    pltpu.sync_copy(zrows_hbm, acc.at[pl.ds(s * ZERO_ROWS, ZERO_ROWS)])
    plsc.subcore_barrier()

    def body(j, carry):
      pltpu.async_copy(xext_hbm.at[src_v.at[j]], rows0, gsem).wait()
      pltpu.sync_copy(rows0, acc.at[dst_v.at[j]], add=True)
      return carry

    lax.fori_loop(0, N_CHUNKS, body, 0)
    plsc.subcore_barrier()

    base = s * ZERO_ROWS
    pltpu.sync_copy(acc.at[pl.ds(base, ZERO_ROWS)],
                    out_hbm.at[c, pl.ds(base, ZERO_ROWS)])

  return k(x_ext, eidx, zrows)


ROWS_BLK = 1000


def _tc_body(p_ref, x_ref, wlt_ref, bl_ref, wrt_ref, wfct_ref, bfc_ref,
             h_ref, probs_ref):
  p = p_ref[0] + p_ref[1]
  summed = p[:, :D_IN]
  cnt = p[:, D_IN:D_IN + 1]
  mean = summed / jnp.maximum(cnt, 1.0)
  h = (jnp.dot(mean, wlt_ref[...], preferred_element_type=jnp.float32)
       + bl_ref[...]
       + jnp.dot(x_ref[...], wrt_ref[...], preferred_element_type=jnp.float32))
  h_ref[...] = h
  logits = jnp.dot(h, wfct_ref[...], preferred_element_type=jnp.float32)
  probs_ref[...] = jax.nn.sigmoid(logits + bfc_ref[...])


def _tc_head(partials, x, W_lT, b_l, W_rT, W_fcT, b_fc2):
  n_blocks = N_NODES // ROWS_BLK
  return pl.pallas_call(
      _tc_body,
      grid=(n_blocks,),
      in_specs=[
          pl.BlockSpec((NC, ROWS_BLK, D_EXT), lambda i: (0, i, 0)),
          pl.BlockSpec((ROWS_BLK, D_IN), lambda i: (i, 0)),
          pl.BlockSpec((D_IN, D_IN), lambda i: (0, 0)),
          pl.BlockSpec((1, D_IN), lambda i: (0, 0)),
          pl.BlockSpec((D_IN, D_IN), lambda i: (0, 0)),
          pl.BlockSpec((D_IN, 1), lambda i: (0, 0)),
          pl.BlockSpec((1, 1), lambda i: (0, 0)),
      ],
      out_specs=[
          pl.BlockSpec((ROWS_BLK, D_IN), lambda i: (i, 0)),
          pl.BlockSpec((ROWS_BLK, 1), lambda i: (i, 0)),
      ],
      out_shape=[
          jax.ShapeDtypeStruct((N_NODES, D_IN), jnp.float32),
          jax.ShapeDtypeStruct((N_NODES, 1), jnp.float32),
      ],
  )(partials, x, W_lT, b_l, W_rT, W_fcT, b_fc2)


def kernel(x, edge_index, W_l, b_l, W_r, W_fc, b_fc):
  x_ext = jnp.concatenate(
      [x, jnp.ones((N_NODES, D_EXT - D_IN), dtype=jnp.float32)], axis=1)
  ei = edge_index.reshape(2, NW, N_EDGES // NW)
  pad_src = jnp.zeros((1, NW, PAD_PER_TILE), dtype=jnp.int32)
  pad_dst = jnp.broadcast_to(
      N_NODES + jnp.arange(PAD_PER_TILE, dtype=jnp.int32) % (ACC_ROWS - N_NODES),
      (1, NW, PAD_PER_TILE))
  eidx = jnp.concatenate([ei, jnp.concatenate([pad_src, pad_dst], 0)], axis=2)
  eidx = eidx.reshape(2, NW, N_CHUNKS, CHUNK)
  zrows = jnp.zeros((ZERO_ROWS, D_EXT), dtype=jnp.float32)

  partials = _sc_segment_sum(x_ext, eidx, zrows)

  h, probs = _tc_head(partials, x, W_l.T, b_l.reshape(1, D_IN),
                      W_r.T, W_fc.T, b_fc.reshape(1, 1))
  return (h, probs.reshape(N_NODES))


# speedup vs baseline: 1.3498x; 1.0076x over previous
import functools

import jax
import jax.numpy as jnp
from jax import lax
from jax.experimental import pallas as pl
from jax.experimental.pallas import tpu as pltpu
from jax.experimental.pallas import tpu_sc as plsc

N_NODES = 10000
N_EDGES = 320000
D_IN = 128
D_EXT = 144

NC = 2
NS = 16
NW = NC * NS

CHUNK = 80
N_CHUNKS = 125
EDGES_PER_TILE = CHUNK * N_CHUNKS
PAD_PER_TILE = EDGES_PER_TILE - N_EDGES // NW

ACC_ROWS = 10112
ZERO_ROWS = ACC_ROWS // NS


def _sc_segment_sum(x_ext, eidx, zrows):
  mesh = plsc.VectorSubcoreMesh(core_axis_name="c", subcore_axis_name="s")

  @functools.partial(
      pl.kernel,
      out_type=jax.ShapeDtypeStruct((NC, ACC_ROWS, D_EXT), jnp.float32),
      mesh=mesh,
      scratch_types=[
          pltpu.VMEM((N_CHUNKS, CHUNK), jnp.int32),
          pltpu.VMEM((N_CHUNKS, CHUNK), jnp.int32),
          pltpu.VMEM((CHUNK, D_EXT), jnp.float32),
          pltpu.VMEM_SHARED((ACC_ROWS, D_EXT), jnp.float32),
          pltpu.SemaphoreType.DMA,
      ],
      compiler_params=pltpu.CompilerParams(use_tc_tiling_on_sc=False),
  )
  def k(xext_hbm, eidx_hbm, zrows_hbm, out_hbm, src_v, dst_v, rows0,
        acc, gsem):
    c = lax.axis_index("c")
    s = lax.axis_index("s")
    w = c * NS + s

    pltpu.sync_copy(eidx_hbm.at[0, w], src_v)
    pltpu.sync_copy(eidx_hbm.at[1, w], dst_v)

    pltpu.sync_copy(zrows_hbm, acc.at[pl.ds(s * ZERO_ROWS, ZERO_ROWS)])
    plsc.subcore_barrier()

    def body(j, carry):
      pltpu.async_copy(xext_hbm.at[src_v.at[j]], rows0, gsem).wait()
      pltpu.sync_copy(rows0, acc.at[dst_v.at[j]], add=True)
      return carry

    lax.fori_loop(0, N_CHUNKS, body, 0)
    plsc.subcore_barrier()

    base = s * ZERO_ROWS
    pltpu.sync_copy(acc.at[pl.ds(base, ZERO_ROWS)],
                    out_hbm.at[c, pl.ds(base, ZERO_ROWS)])

  return k(x_ext, eidx, zrows)


ROWS_BLK = 2000


def _tc_body(p_ref, x_ref, wlt_ref, bl_ref, wrt_ref, wfct_ref, bfc_ref,
             h_ref, probs_ref):
  p = p_ref[0] + p_ref[1]
  summed = p[:, :D_IN]
  cnt = p[:, D_IN:D_IN + 1]
  mean = summed / jnp.maximum(cnt, 1.0)
  h = (jnp.dot(mean, wlt_ref[...], preferred_element_type=jnp.float32)
       + bl_ref[...]
       + jnp.dot(x_ref[...], wrt_ref[...], preferred_element_type=jnp.float32))
  h_ref[...] = h
  logits = jnp.dot(h, wfct_ref[...], preferred_element_type=jnp.float32)
  probs_ref[...] = jax.nn.sigmoid(logits + bfc_ref[...])


def _tc_head(partials, x, W_lT, b_l, W_rT, W_fcT, b_fc2):
  n_blocks = N_NODES // ROWS_BLK
  return pl.pallas_call(
      _tc_body,
      grid=(n_blocks,),
      in_specs=[
          pl.BlockSpec((NC, ROWS_BLK, D_EXT), lambda i: (0, i, 0)),
          pl.BlockSpec((ROWS_BLK, D_IN), lambda i: (i, 0)),
          pl.BlockSpec((D_IN, D_IN), lambda i: (0, 0)),
          pl.BlockSpec((1, D_IN), lambda i: (0, 0)),
          pl.BlockSpec((D_IN, D_IN), lambda i: (0, 0)),
          pl.BlockSpec((D_IN, 1), lambda i: (0, 0)),
          pl.BlockSpec((1, 1), lambda i: (0, 0)),
      ],
      out_specs=[
          pl.BlockSpec((ROWS_BLK, D_IN), lambda i: (i, 0)),
          pl.BlockSpec((ROWS_BLK, 1), lambda i: (i, 0)),
      ],
      out_shape=[
          jax.ShapeDtypeStruct((N_NODES, D_IN), jnp.float32),
          jax.ShapeDtypeStruct((N_NODES, 1), jnp.float32),
      ],
  )(partials, x, W_lT, b_l, W_rT, W_fcT, b_fc2)


def kernel(x, edge_index, W_l, b_l, W_r, W_fc, b_fc):
  x_ext = jnp.concatenate(
      [x, jnp.ones((N_NODES, D_EXT - D_IN), dtype=jnp.float32)], axis=1)
  ei = edge_index.reshape(2, NW, N_EDGES // NW)
  pad_src = jnp.zeros((1, NW, PAD_PER_TILE), dtype=jnp.int32)
  pad_dst = jnp.broadcast_to(
      N_NODES + jnp.arange(PAD_PER_TILE, dtype=jnp.int32) % (ACC_ROWS - N_NODES),
      (1, NW, PAD_PER_TILE))
  eidx = jnp.concatenate([ei, jnp.concatenate([pad_src, pad_dst], 0)], axis=2)
  eidx = eidx.reshape(2, NW, N_CHUNKS, CHUNK)
  zrows = jnp.zeros((ZERO_ROWS, D_EXT), dtype=jnp.float32)

  partials = _sc_segment_sum(x_ext, eidx, zrows)

  h, probs = _tc_head(partials, x, W_l.T, b_l.reshape(1, D_IN),
                      W_r.T, W_fc.T, b_fc.reshape(1, 1))
  return (h, probs.reshape(N_NODES))


# speedup vs baseline: 1.3585x; 1.0065x over previous
import functools

import jax
import jax.numpy as jnp
from jax import lax
from jax.experimental import pallas as pl
from jax.experimental.pallas import tpu as pltpu
from jax.experimental.pallas import tpu_sc as plsc

N_NODES = 10000
N_EDGES = 320000
D_IN = 128
D_EXT = 144

NC = 2
NS = 16
NW = NC * NS

CHUNK = 80
N_CHUNKS = 125
EDGES_PER_TILE = CHUNK * N_CHUNKS
PAD_PER_TILE = EDGES_PER_TILE - N_EDGES // NW

ACC_ROWS = 10112
ZERO_ROWS = ACC_ROWS // NS


def _sc_segment_sum(x_ext, eidx, zrows):
  mesh = plsc.VectorSubcoreMesh(core_axis_name="c", subcore_axis_name="s")

  @functools.partial(
      pl.kernel,
      out_type=jax.ShapeDtypeStruct((NC, ACC_ROWS, D_EXT), jnp.float32),
      mesh=mesh,
      scratch_types=[
          pltpu.VMEM((N_CHUNKS, CHUNK), jnp.int32),
          pltpu.VMEM((N_CHUNKS, CHUNK), jnp.int32),
          pltpu.VMEM((CHUNK, D_EXT), jnp.float32),
          pltpu.VMEM_SHARED((ACC_ROWS, D_EXT), jnp.float32),
          pltpu.SemaphoreType.DMA,
      ],
      compiler_params=pltpu.CompilerParams(use_tc_tiling_on_sc=False),
  )
  def k(xext_hbm, eidx_hbm, zrows_hbm, out_hbm, src_v, dst_v, rows0,
        acc, gsem):
    c = lax.axis_index("c")
    s = lax.axis_index("s")
    w = c * NS + s

    d0 = pltpu.async_copy(eidx_hbm.at[0, w], src_v, gsem)
    d1 = pltpu.async_copy(eidx_hbm.at[1, w], dst_v, gsem)
    d2 = pltpu.async_copy(zrows_hbm, acc.at[pl.ds(s * ZERO_ROWS, ZERO_ROWS)],
                          gsem)
    d0.wait()
    d1.wait()
    d2.wait()
    plsc.subcore_barrier()

    def body(j, carry):
      pltpu.async_copy(xext_hbm.at[src_v.at[j]], rows0, gsem).wait()
      pltpu.sync_copy(rows0, acc.at[dst_v.at[j]], add=True)
      return carry

    lax.fori_loop(0, N_CHUNKS, body, 0, unroll=5)
    plsc.subcore_barrier()

    base = s * ZERO_ROWS
    pltpu.sync_copy(acc.at[pl.ds(base, ZERO_ROWS)],
                    out_hbm.at[c, pl.ds(base, ZERO_ROWS)])

  return k(x_ext, eidx, zrows)


ROWS_BLK = 2000


def _tc_body(p_ref, x_ref, wlt_ref, bl_ref, wrt_ref, wfct_ref, bfc_ref,
             h_ref, probs_ref):
  p = p_ref[0] + p_ref[1]
  summed = p[:, :D_IN]
  cnt = p[:, D_IN:D_IN + 1]
  mean = summed / jnp.maximum(cnt, 1.0)
  h = (jnp.dot(mean, wlt_ref[...], preferred_element_type=jnp.float32)
       + bl_ref[...]
       + jnp.dot(x_ref[...], wrt_ref[...], preferred_element_type=jnp.float32))
  h_ref[...] = h
  logits = jnp.dot(h, wfct_ref[...], preferred_element_type=jnp.float32)
  probs_ref[...] = jax.nn.sigmoid(logits + bfc_ref[...])


def _tc_head(partials, x, W_lT, b_l, W_rT, W_fcT, b_fc2):
  n_blocks = N_NODES // ROWS_BLK
  return pl.pallas_call(
      _tc_body,
      grid=(n_blocks,),
      in_specs=[
          pl.BlockSpec((NC, ROWS_BLK, D_EXT), lambda i: (0, i, 0)),
          pl.BlockSpec((ROWS_BLK, D_IN), lambda i: (i, 0)),
          pl.BlockSpec((D_IN, D_IN), lambda i: (0, 0)),
          pl.BlockSpec((1, D_IN), lambda i: (0, 0)),
          pl.BlockSpec((D_IN, D_IN), lambda i: (0, 0)),
          pl.BlockSpec((D_IN, 1), lambda i: (0, 0)),
          pl.BlockSpec((1, 1), lambda i: (0, 0)),
      ],
      out_specs=[
          pl.BlockSpec((ROWS_BLK, D_IN), lambda i: (i, 0)),
          pl.BlockSpec((ROWS_BLK, 1), lambda i: (i, 0)),
      ],
      out_shape=[
          jax.ShapeDtypeStruct((N_NODES, D_IN), jnp.float32),
          jax.ShapeDtypeStruct((N_NODES, 1), jnp.float32),
      ],
  )(partials, x, W_lT, b_l, W_rT, W_fcT, b_fc2)


def kernel(x, edge_index, W_l, b_l, W_r, W_fc, b_fc):
  x_ext = jnp.concatenate(
      [x, jnp.ones((N_NODES, D_EXT - D_IN), dtype=jnp.float32)], axis=1)
  ei = edge_index.reshape(2, NW, N_EDGES // NW)
  pad_src = jnp.zeros((1, NW, PAD_PER_TILE), dtype=jnp.int32)
  pad_dst = jnp.broadcast_to(
      N_NODES + jnp.arange(PAD_PER_TILE, dtype=jnp.int32) % (ACC_ROWS - N_NODES),
      (1, NW, PAD_PER_TILE))
  eidx = jnp.concatenate([ei, jnp.concatenate([pad_src, pad_dst], 0)], axis=2)
  eidx = eidx.reshape(2, NW, N_CHUNKS, CHUNK)
  zrows = jnp.zeros((ZERO_ROWS, D_EXT), dtype=jnp.float32)

  partials = _sc_segment_sum(x_ext, eidx, zrows)

  h, probs = _tc_head(partials, x, W_l.T, b_l.reshape(1, D_IN),
                      W_r.T, W_fc.T, b_fc.reshape(1, 1))
  return (h, probs.reshape(N_NODES))
